# Initial kernel scaffold; baseline (speedup 1.0000x reference)
import jax
import jax.numpy as jnp
from jax.experimental import pallas as pl


def kernel(x, Ws1, bs1, Ws2, bs2, Wr1, br1, Wr2, br2, Wg, bg):
    raise NotImplementedError("write your pallas kernel here")


# speedup vs baseline: 5.0557x; 5.0557x over previous
import functools

import jax
import jax.numpy as jnp
from jax.experimental import pallas as pl

D_MODEL = 1024
D_HIDDEN = 256
N_SHARED = 2
N_ROUTED = 16
T_BLOCK = 256


def _dense_body(x_ref, Ws1_ref, bs1_ref, Ws2_ref, bs2_ref,
                Wr1_ref, br1_ref, Wr2_ref, br2_ref, Wg_ref, bg_ref, out_ref):
    xb = x_ref[...]
    acc = jnp.zeros_like(xb)
    for s in range(N_SHARED):
        h = jnp.maximum(
            jnp.dot(xb, Ws1_ref[s], preferred_element_type=jnp.float32)
            + bs1_ref[s][None, :], 0.0)
        acc = acc + jnp.dot(h, Ws2_ref[s], preferred_element_type=jnp.float32) \
            + bs2_ref[s][None, :]
    acc = acc * (1.0 / N_SHARED)

    logits = jnp.dot(xb, Wg_ref[...], preferred_element_type=jnp.float32) \
        + bg_ref[...][None, :]
    iota = jax.lax.broadcasted_iota(jnp.int32, logits.shape, 1)
    m1 = jnp.max(logits, axis=-1, keepdims=True)
    i1 = jnp.min(jnp.where(logits == m1, iota, N_ROUTED), axis=-1, keepdims=True)
    masked = jnp.where(iota == i1, -jnp.inf, logits)
    m2 = jnp.max(masked, axis=-1, keepdims=True)
    i2 = jnp.min(jnp.where(masked == m2, iota, N_ROUTED), axis=-1, keepdims=True)
    w1 = 1.0 / (1.0 + jnp.exp(m2 - m1))
    w2 = 1.0 - w1
    gate = jnp.where(iota == i1, w1, 0.0) + jnp.where(iota == i2, w2, 0.0)

    for e in range(N_ROUTED):
        h = jnp.maximum(
            jnp.dot(xb, Wr1_ref[e], preferred_element_type=jnp.float32)
            + br1_ref[e][None, :], 0.0)
        y = jnp.dot(h, Wr2_ref[e], preferred_element_type=jnp.float32) \
            + br2_ref[e][None, :]
        acc = acc + y * gate[:, e:e + 1]
    out_ref[...] = acc


def _full(shape):
    return pl.BlockSpec(shape, lambda i: tuple(0 for _ in shape))


@functools.partial(jax.jit, static_argnames=("interpret",))
def _moe_dense(flat, Ws1, bs1, Ws2, bs2, Wr1, br1, Wr2, br2, Wg, bg,
               interpret=False):
    T = flat.shape[0]
    grid = (T // T_BLOCK,)
    return pl.pallas_call(
        _dense_body,
        grid=grid,
        in_specs=[
            pl.BlockSpec((T_BLOCK, D_MODEL), lambda i: (i, 0)),
            _full(Ws1.shape), _full(bs1.shape),
            _full(Ws2.shape), _full(bs2.shape),
            _full(Wr1.shape), _full(br1.shape),
            _full(Wr2.shape), _full(br2.shape),
            _full(Wg.shape), _full(bg.shape),
        ],
        out_specs=pl.BlockSpec((T_BLOCK, D_MODEL), lambda i: (i, 0)),
        out_shape=jax.ShapeDtypeStruct((T, D_MODEL), jnp.float32),
        interpret=interpret,
    )(flat, Ws1, bs1, Ws2, bs2, Wr1, br1, Wr2, br2, Wg, bg)


def kernel(x, Ws1, bs1, Ws2, bs2, Wr1, br1, Wr2, br2, Wg, bg):
    B, S, D = x.shape
    flat = x.reshape(-1, D)
    out = _moe_dense(flat, Ws1, bs1, Ws2, bs2, Wr1, br1, Wr2, br2, Wg, bg)
    return out.reshape(B, S, D)
